# Initial kernel scaffold; baseline (speedup 1.0000x reference)
import jax
import jax.numpy as jnp
from jax.experimental import pallas as pl


def kernel(user_ids, item_ids, user_table, item_table):
    raise NotImplementedError("write your pallas kernel here")


# speedup vs baseline: 1.4930x; 1.4930x over previous
import functools

import jax
import jax.numpy as jnp
from jax import lax
from jax.experimental import pallas as pl
from jax.experimental.pallas import tpu as pltpu
from jax.experimental.pallas import tpu_sc as plsc

BATCH = 16384
D = 128
NC = 2
NS = 16
NW = NC * NS
BPW = BATCH // NW
CH = 128
NCH = BPW // CH
PH = 2
CPP = NCH // PH
RPP = BPW // PH

_mesh = plsc.VectorSubcoreMesh(core_axis_name="c", subcore_axis_name="s")


@functools.partial(
    pl.kernel,
    mesh=_mesh,
    out_type=(
        jax.ShapeDtypeStruct((BATCH, D), jnp.float32),
        jax.ShapeDtypeStruct((BATCH, D), jnp.float32),
    ),
    scratch_types=[
        pltpu.VMEM((NCH, CH), jnp.int32),
        pltpu.VMEM((NCH, CH), jnp.int32),
        pltpu.VMEM((RPP, D), jnp.float32),
        pltpu.VMEM((RPP, D), jnp.float32),
        pltpu.SemaphoreType.DMA,
    ],
)
def _sc_gather(uids, iids, utab, itab, uout, iout, uidx_v, iidx_v,
               urows_v, irows_v, sem):
    wid = lax.axis_index("s") * NC + lax.axis_index("c")
    base = wid * BPW
    pltpu.sync_copy(uids.at[wid], uidx_v)
    pltpu.sync_copy(iids.at[wid], iidx_v)
    for h in range(PH):
        copies = []
        for j in range(CPP):
            ch = h * CPP + j
            copies.append(pltpu.async_copy(
                utab.at[uidx_v.at[ch]], urows_v.at[pl.ds(j * CH, CH)], sem))
            copies.append(pltpu.async_copy(
                itab.at[iidx_v.at[ch]], irows_v.at[pl.ds(j * CH, CH)], sem))
        for c in copies:
            c.wait()
        pltpu.sync_copy(urows_v, uout.at[pl.ds(base + h * RPP, RPP)])
        pltpu.sync_copy(irows_v, iout.at[pl.ds(base + h * RPP, RPP)])


def kernel(user_ids, item_ids, user_table, item_table):
    uids = user_ids.astype(jnp.int32).reshape(NW, NCH, CH)
    iids = item_ids.astype(jnp.int32).reshape(NW, NCH, CH)
    return _sc_gather(uids, iids, user_table, item_table)


# speedup vs baseline: 1.5589x; 1.0441x over previous
import functools

import jax
import jax.numpy as jnp
from jax import lax
from jax.experimental import pallas as pl
from jax.experimental.pallas import tpu as pltpu
from jax.experimental.pallas import tpu_sc as plsc

BATCH = 16384
D = 128
NC = 2
NS = 16
NW = NC * NS
BPW = BATCH // NW
CH = 128
NCH = BPW // CH
NCHUNKS = 2 * NCH
NBUF = 7

_mesh = plsc.VectorSubcoreMesh(core_axis_name="c", subcore_axis_name="s")


@functools.partial(
    pl.kernel,
    mesh=_mesh,
    out_type=(
        jax.ShapeDtypeStruct((BATCH, D), jnp.float32),
        jax.ShapeDtypeStruct((BATCH, D), jnp.float32),
    ),
    scratch_types=[
        pltpu.VMEM((NCH, CH), jnp.int32),
        pltpu.VMEM((NCH, CH), jnp.int32),
        pltpu.VMEM((NBUF, CH, D), jnp.float32),
        pltpu.SemaphoreType.DMA((NCHUNKS,)),
        pltpu.SemaphoreType.DMA((NCHUNKS,)),
    ],
)
def _sc_gather(uids, iids, utab, itab, uout, iout, uidx_v, iidx_v,
               bufs, gsem, wsem):
    wid = lax.axis_index("s") * NC + lax.axis_index("c")
    base = wid * BPW
    pltpu.sync_copy(uids.at[wid], uidx_v)
    pltpu.sync_copy(iids.at[wid], iidx_v)

    def chunk(c):
        j = c // 2
        if c % 2 == 0:
            return utab, uidx_v.at[j], uout, base + j * CH
        return itab, iidx_v.at[j], iout, base + j * CH

    gathers = []
    for c in range(NCHUNKS):
        tab, idx, _, _ = chunk(c)
        if c < NBUF:
            gathers.append(pltpu.async_copy(tab.at[idx], bufs.at[c],
                                            gsem.at[c]))
        else:
            gathers.append(None)

    writebacks = []
    for c in range(NCHUNKS):
        _, _, out, off = chunk(c)
        b = c % NBUF
        gathers[c].wait()
        writebacks.append(pltpu.async_copy(bufs.at[b],
                                           out.at[pl.ds(off, CH)], wsem.at[c]))
        if c + NBUF < NCHUNKS:
            writebacks[c].wait()
            writebacks[c] = None
            tab, idx, _, _ = chunk(c + NBUF)
            gathers[c + NBUF] = pltpu.async_copy(tab.at[idx], bufs.at[b],
                                                 gsem.at[c + NBUF])

    for w in writebacks:
        if w is not None:
            w.wait()


def kernel(user_ids, item_ids, user_table, item_table):
    uids = user_ids.astype(jnp.int32).reshape(NW, NCH, CH)
    iids = item_ids.astype(jnp.int32).reshape(NW, NCH, CH)
    return _sc_gather(uids, iids, user_table, item_table)
